# baseline (device time: 3036569 ns/iter reference)
import jax
import jax.numpy as jnp
from jax import lax
from jax.experimental import pallas as pl
from jax.experimental.pallas import tpu as pltpu


def _peer():
    return (lax.axis_index("x"), 1 - lax.axis_index("y"), lax.axis_index("z"))


def _exchange_kernel(xb, a2):
    T, D = xb.shape

    def body(x_ref, a_ref, xall_ref, aall_ref, cx_sem, ca_sem, sx_send, sx_recv, sa_send, sa_recv):
        peer = _peer()
        cx = pltpu.make_async_copy(x_ref, xall_ref.at[0], cx_sem)
        ca = pltpu.make_async_copy(a_ref, aall_ref.at[0], ca_sem)
        cx.start()
        ca.start()
        rx = pltpu.make_async_remote_copy(
            src_ref=x_ref,
            dst_ref=xall_ref.at[1],
            send_sem=sx_send,
            recv_sem=sx_recv,
            device_id=peer,
            device_id_type=pl.DeviceIdType.MESH,
        )
        ra = pltpu.make_async_remote_copy(
            src_ref=a_ref,
            dst_ref=aall_ref.at[1],
            send_sem=sa_send,
            recv_sem=sa_recv,
            device_id=peer,
            device_id_type=pl.DeviceIdType.MESH,
        )
        rx.start()
        ra.start()
        cx.wait()
        ca.wait()
        rx.wait()
        ra.wait()

    return pl.pallas_call(
        body,
        out_shape=(
            jax.ShapeDtypeStruct((2, T, D), jnp.bfloat16),
            jax.ShapeDtypeStruct((2, T, 1), jnp.int32),
        ),
        in_specs=[
            pl.BlockSpec(memory_space=pltpu.MemorySpace.VMEM),
            pl.BlockSpec(memory_space=pltpu.MemorySpace.VMEM),
        ],
        out_specs=(
            pl.BlockSpec(memory_space=pltpu.MemorySpace.HBM),
            pl.BlockSpec(memory_space=pltpu.MemorySpace.HBM),
        ),
        scratch_shapes=[
            pltpu.SemaphoreType.DMA,
            pltpu.SemaphoreType.DMA,
            pltpu.SemaphoreType.DMA,
            pltpu.SemaphoreType.DMA,
            pltpu.SemaphoreType.DMA,
            pltpu.SemaphoreType.DMA,
        ],
    )(xb, a2)


def _moe_kernel(xall, aall, W1b, W2b, MT=512, FT=1024):
    _, T, D = xall.shape
    E_loc, _, F = W1b.shape
    n_m = 2 * T // MT
    n_f = F // FT
    tiles_per_slot = T // MT

    def body(x_ref, a_ref, w1_ref, w2_ref, out_ref, acc_ref):
        e = pl.program_id(1)
        f = pl.program_id(2)
        my_y = lax.axis_index("y")
        ge = my_y * E_loc + e

        mask = a_ref[0] == ge
        xm = jnp.where(mask, x_ref[0], jnp.bfloat16(0))
        h = jnp.dot(xm, w1_ref[0], preferred_element_type=jnp.float32)
        h = jnp.maximum(h, 0.0).astype(jnp.bfloat16)
        part = jnp.dot(h, w2_ref[0], preferred_element_type=jnp.float32)

        first = jnp.logical_and(e == 0, f == 0)
        last = jnp.logical_and(e == E_loc - 1, f == n_f - 1)

        @pl.when(first)
        def _():
            acc_ref[...] = part

        @pl.when(jnp.logical_not(first))
        def _():
            acc_ref[...] += part

        @pl.when(last)
        def _():
            out_ref[0] = acc_ref[...].astype(jnp.bfloat16)

    return pl.pallas_call(
        body,
        grid=(n_m, E_loc, n_f),
        out_shape=jax.ShapeDtypeStruct((2, T, D), jnp.bfloat16),
        in_specs=[
            pl.BlockSpec(
                (1, MT, D), lambda m, e, f: (m // tiles_per_slot, m % tiles_per_slot, 0)
            ),
            pl.BlockSpec(
                (1, MT, 1), lambda m, e, f: (m // tiles_per_slot, m % tiles_per_slot, 0)
            ),
            pl.BlockSpec((1, D, FT), lambda m, e, f: (e, 0, f)),
            pl.BlockSpec((1, FT, D), lambda m, e, f: (e, f, 0)),
        ],
        out_specs=pl.BlockSpec(
            (1, MT, D), lambda m, e, f: (m // tiles_per_slot, m % tiles_per_slot, 0)
        ),
        scratch_shapes=[pltpu.VMEM((MT, D), jnp.float32)],
        compiler_params=pltpu.CompilerParams(
            dimension_semantics=("arbitrary", "arbitrary", "arbitrary"),
        ),
    )(xall, aall, W1b, W2b)


def _grouped_moe_kernel(xg, W1b, W2b, C, MT=512, FT=1024):
    R, D = xg.shape
    E_loc, _, F = W1b.shape
    n_m = R // MT
    n_f = F // FT
    tiles_per_e = C // MT

    def body(x_ref, w1_ref, w2_ref, out_ref, acc_ref):
        f = pl.program_id(1)
        h = jnp.dot(x_ref[...], w1_ref[0], preferred_element_type=jnp.float32)
        h = jnp.maximum(h, 0.0).astype(jnp.bfloat16)
        part = jnp.dot(h, w2_ref[0], preferred_element_type=jnp.float32)

        @pl.when(f == 0)
        def _():
            acc_ref[...] = part

        @pl.when(f != 0)
        def _():
            acc_ref[...] += part

        @pl.when(f == n_f - 1)
        def _():
            out_ref[...] = acc_ref[...].astype(jnp.bfloat16)

    return pl.pallas_call(
        body,
        grid=(n_m, n_f),
        out_shape=jax.ShapeDtypeStruct((R, D), jnp.bfloat16),
        in_specs=[
            pl.BlockSpec((MT, D), lambda m, f: (m, 0)),
            pl.BlockSpec((1, D, FT), lambda m, f: (m // tiles_per_e, 0, f)),
            pl.BlockSpec((1, FT, D), lambda m, f: (m // tiles_per_e, f, 0)),
        ],
        out_specs=pl.BlockSpec((MT, D), lambda m, f: (m, 0)),
        scratch_shapes=[pltpu.VMEM((MT, D), jnp.float32)],
        compiler_params=pltpu.CompilerParams(
            dimension_semantics=("arbitrary", "arbitrary"),
        ),
    )(xg, W1b, W2b)


def _combine_kernel(contrib):
    _, T, D = contrib.shape

    def body(c_ref, out_ref, recv_ref, copy_sem, send_sem, recv_sem):
        peer = _peer()
        local = pltpu.make_async_copy(c_ref.at[0], out_ref, copy_sem)
        local.start()
        rdma = pltpu.make_async_remote_copy(
            src_ref=c_ref.at[1],
            dst_ref=recv_ref,
            send_sem=send_sem,
            recv_sem=recv_sem,
            device_id=peer,
            device_id_type=pl.DeviceIdType.MESH,
        )
        rdma.start()
        local.wait()
        rdma.wait()
        out_ref[...] = out_ref[...] + recv_ref[...]

    return pl.pallas_call(
        body,
        out_shape=jax.ShapeDtypeStruct((T, D), jnp.bfloat16),
        in_specs=[pl.BlockSpec(memory_space=pltpu.MemorySpace.HBM)],
        out_specs=pl.BlockSpec(memory_space=pltpu.MemorySpace.VMEM),
        scratch_shapes=[
            pltpu.VMEM((T, D), jnp.bfloat16),
            pltpu.SemaphoreType.DMA,
            pltpu.SemaphoreType.DMA,
            pltpu.SemaphoreType.DMA,
        ],
        compiler_params=pltpu.CompilerParams(vmem_limit_bytes=50 * 2**20),
    )(contrib)


def kernel(x, assign, W1, W2):
    T, D = x.shape
    E_loc = W1.shape[0]
    C = 1536
    xb = x.astype(jnp.bfloat16)
    a2 = assign.reshape(T, 1)
    W1b = W1.astype(jnp.bfloat16)
    W2b = W2.astype(jnp.bfloat16)

    xall, aall = _exchange_kernel(xb, a2)

    N = 2 * T
    R = E_loc * C
    af = aall.reshape(N)
    ge_base = E_loc * lax.axis_index("y")
    oh = af[:, None] == (ge_base + jnp.arange(E_loc, dtype=jnp.int32))[None, :]
    ohi = oh.astype(jnp.int32)
    rank = ((jnp.cumsum(ohi, axis=0) - ohi) * ohi).sum(axis=1)
    mine = oh.any(axis=1)
    le = af - ge_base
    dest = jnp.where(mine, le * C + rank, R).astype(jnp.int32)
    key = jnp.where(mine, le, E_loc).astype(jnp.int32)
    g = jnp.argsort(key, stable=True).astype(jnp.int32)
    counts = ohi.sum(axis=0)
    offsets = jnp.concatenate(
        [jnp.zeros((1,), jnp.int32), jnp.cumsum(counts)[:-1].astype(jnp.int32)]
    )
    s = jnp.arange(R, dtype=jnp.int32)
    e_s = s // C
    r_s = s % C
    src_pos = offsets[e_s] + r_s
    valid = r_s < counts[e_s]
    token_idx = jnp.take(g, src_pos, mode="fill", fill_value=N)
    bucket = jnp.where(valid, token_idx, N)
    xg = jnp.take(xall.reshape(N, D), bucket, axis=0, mode="fill", fill_value=0)

    yg = _grouped_moe_kernel(xg, W1b, W2b, C)

    contrib = jnp.take(yg, dest, axis=0, mode="fill", fill_value=0).reshape(2, T, D)
    out = _combine_kernel(contrib)
    return out.astype(jnp.float32)


# device time: 566815 ns/iter; 5.3572x vs baseline; 5.3572x over previous
import jax
import jax.numpy as jnp
from jax import lax
from jax.experimental import pallas as pl
from jax.experimental.pallas import tpu as pltpu

N_RING = 8


def _peer():
    return (lax.axis_index("x"), 1 - lax.axis_index("y"), lax.axis_index("z"))


def _ring_pos():
    x = lax.axis_index("x")
    z = lax.axis_index("z")
    return jnp.where(x == 0, z, 2 * N_RING // 2 - 1 - z)


def _ring_coords(t):
    x = jnp.where(t < N_RING // 2, 0, 1)
    z = jnp.where(t < N_RING // 2, t, N_RING - 1 - t)
    return x, z


def _exchange_kernel(xsl, asl):
    S, D = xsl.shape

    def body(x_ref, a_ref, xloc_ref, aloc_ref, sx_send, sx_recv, sa_send, sa_recv):
        peer = _peer()
        xloc_ref[0] = x_ref[...]
        aloc_ref[0] = a_ref[...]
        rx = pltpu.make_async_remote_copy(
            src_ref=x_ref,
            dst_ref=xloc_ref.at[1],
            send_sem=sx_send,
            recv_sem=sx_recv,
            device_id=peer,
            device_id_type=pl.DeviceIdType.MESH,
        )
        ra = pltpu.make_async_remote_copy(
            src_ref=a_ref,
            dst_ref=aloc_ref.at[1],
            send_sem=sa_send,
            recv_sem=sa_recv,
            device_id=peer,
            device_id_type=pl.DeviceIdType.MESH,
        )
        rx.start()
        ra.start()
        rx.wait()
        ra.wait()

    return pl.pallas_call(
        body,
        out_shape=(
            jax.ShapeDtypeStruct((2, S, D), jnp.bfloat16),
            jax.ShapeDtypeStruct((2, S, 1), jnp.int32),
        ),
        in_specs=[
            pl.BlockSpec(memory_space=pltpu.MemorySpace.VMEM),
            pl.BlockSpec(memory_space=pltpu.MemorySpace.VMEM),
        ],
        out_specs=(
            pl.BlockSpec(memory_space=pltpu.MemorySpace.VMEM),
            pl.BlockSpec(memory_space=pltpu.MemorySpace.VMEM),
        ),
        scratch_shapes=[
            pltpu.SemaphoreType.DMA,
            pltpu.SemaphoreType.DMA,
            pltpu.SemaphoreType.DMA,
            pltpu.SemaphoreType.DMA,
        ],
    )(xsl, asl)


def _moe_kernel(xall, aall, W1b, W2b, MT=512, FT=1024):
    _, T, D = xall.shape
    E_loc, _, F = W1b.shape
    n_m = 2 * T // MT
    n_f = F // FT
    tiles_per_slot = T // MT

    def body(x_ref, a_ref, w1_ref, w2_ref, out_ref, acc_ref):
        e = pl.program_id(1)
        f = pl.program_id(2)
        my_y = lax.axis_index("y")
        ge = my_y * E_loc + e

        mask = a_ref[0] == ge
        xm = jnp.where(mask, x_ref[0], jnp.bfloat16(0))
        h = jnp.dot(xm, w1_ref[0], preferred_element_type=jnp.float32)
        h = jnp.maximum(h, 0.0).astype(jnp.bfloat16)
        part = jnp.dot(h, w2_ref[0], preferred_element_type=jnp.float32)

        first = jnp.logical_and(e == 0, f == 0)
        last = jnp.logical_and(e == E_loc - 1, f == n_f - 1)

        @pl.when(first)
        def _():
            acc_ref[...] = part

        @pl.when(jnp.logical_not(first))
        def _():
            acc_ref[...] += part

        @pl.when(last)
        def _():
            out_ref[0] = acc_ref[...].astype(jnp.bfloat16)

    return pl.pallas_call(
        body,
        grid=(n_m, E_loc, n_f),
        out_shape=jax.ShapeDtypeStruct((2, T, D), jnp.bfloat16),
        in_specs=[
            pl.BlockSpec(
                (1, MT, D), lambda m, e, f: (m // tiles_per_slot, m % tiles_per_slot, 0)
            ),
            pl.BlockSpec(
                (1, MT, 1), lambda m, e, f: (m // tiles_per_slot, m % tiles_per_slot, 0)
            ),
            pl.BlockSpec((1, D, FT), lambda m, e, f: (e, 0, f)),
            pl.BlockSpec((1, FT, D), lambda m, e, f: (e, f, 0)),
        ],
        out_specs=pl.BlockSpec(
            (1, MT, D), lambda m, e, f: (m // tiles_per_slot, m % tiles_per_slot, 0)
        ),
        scratch_shapes=[pltpu.VMEM((MT, D), jnp.float32)],
        compiler_params=pltpu.CompilerParams(
            dimension_semantics=("arbitrary", "arbitrary", "arbitrary"),
        ),
    )(xall, aall, W1b, W2b)


def _combine_kernel(contrib):
    _, S, D = contrib.shape

    def body(c_ref, out_ref, recv_ref, copy_sem, send_sem, recv_sem):
        peer = _peer()
        local = pltpu.make_async_copy(c_ref.at[0], out_ref, copy_sem)
        local.start()
        rdma = pltpu.make_async_remote_copy(
            src_ref=c_ref.at[1],
            dst_ref=recv_ref,
            send_sem=send_sem,
            recv_sem=recv_sem,
            device_id=peer,
            device_id_type=pl.DeviceIdType.MESH,
        )
        rdma.start()
        local.wait()
        rdma.wait()
        out_ref[...] = out_ref[...] + recv_ref[...]

    return pl.pallas_call(
        body,
        out_shape=jax.ShapeDtypeStruct((S, D), jnp.bfloat16),
        in_specs=[pl.BlockSpec(memory_space=pltpu.MemorySpace.HBM)],
        out_specs=pl.BlockSpec(memory_space=pltpu.MemorySpace.VMEM),
        scratch_shapes=[
            pltpu.VMEM((S, D), jnp.bfloat16),
            pltpu.SemaphoreType.DMA,
            pltpu.SemaphoreType.DMA,
            pltpu.SemaphoreType.DMA,
        ],
    )(contrib)


def _allgather_kernel(chunk):
    S, D = chunk.shape

    def body(c_ref, out_ref, comm_ref, send_sems, recv_sems):
        rp = _ring_pos()
        my_y = lax.axis_index("y")
        rx, rz = _ring_coords((rp + 1) % N_RING)
        right = (rx, my_y, rz)

        out_ref[pl.ds(rp * S, S), :] = c_ref[...]
        comm_ref[0] = c_ref[...]

        for h in range(N_RING - 1):
            send_slot = h % 2
            recv_slot = (h + 1) % 2
            rdma = pltpu.make_async_remote_copy(
                src_ref=comm_ref.at[send_slot],
                dst_ref=comm_ref.at[recv_slot],
                send_sem=send_sems.at[send_slot],
                recv_sem=recv_sems.at[recv_slot],
                device_id=right,
                device_id_type=pl.DeviceIdType.MESH,
            )
            rdma.start()
            rdma.wait()
            origin = (rp - h - 1) % N_RING
            out_ref[pl.ds(origin * S, S), :] = comm_ref[recv_slot]

    return pl.pallas_call(
        body,
        out_shape=jax.ShapeDtypeStruct((N_RING * S, D), jnp.bfloat16),
        in_specs=[pl.BlockSpec(memory_space=pltpu.MemorySpace.VMEM)],
        out_specs=pl.BlockSpec(memory_space=pltpu.MemorySpace.VMEM),
        scratch_shapes=[
            pltpu.VMEM((2, S, D), jnp.bfloat16),
            pltpu.SemaphoreType.DMA((2,)),
            pltpu.SemaphoreType.DMA((2,)),
        ],
        compiler_params=pltpu.CompilerParams(vmem_limit_bytes=48 * 2**20),
    )(chunk)


def kernel(x, assign, W1, W2):
    T, D = x.shape
    S = T // N_RING
    xb = x.astype(jnp.bfloat16)
    a2 = assign.reshape(T, 1)
    W1b = W1.astype(jnp.bfloat16)
    W2b = W2.astype(jnp.bfloat16)

    rp = _ring_pos()
    xsl = lax.dynamic_slice(xb, (rp * S, 0), (S, D))
    asl = lax.dynamic_slice(a2, (rp * S, 0), (S, 1))

    xloc, aloc = _exchange_kernel(xsl, asl)
    contrib = _moe_kernel(xloc, aloc, W1b, W2b)
    mine = _combine_kernel(contrib)
    out = _allgather_kernel(mine)
    return out.astype(jnp.float32)


# device time: 398368 ns/iter; 7.6225x vs baseline; 1.4228x over previous
import jax
import jax.numpy as jnp
from jax import lax
from jax.experimental import pallas as pl
from jax.experimental.pallas import tpu as pltpu

N_RING = 8


def _peer():
    return (lax.axis_index("x"), 1 - lax.axis_index("y"), lax.axis_index("z"))


def _ring_pos():
    x = lax.axis_index("x")
    z = lax.axis_index("z")
    return jnp.where(x == 0, z, 2 * N_RING // 2 - 1 - z)


def _ring_coords(t):
    x = jnp.where(t < N_RING // 2, 0, 1)
    z = jnp.where(t < N_RING // 2, t, N_RING - 1 - t)
    return x, z


def _exchange_kernel(xsl, asl):
    S, D = xsl.shape

    def body(x_ref, a_ref, xloc_ref, aloc_ref, sx_send, sx_recv, sa_send, sa_recv):
        peer = _peer()
        xloc_ref[0] = x_ref[...]
        aloc_ref[0] = a_ref[...]
        rx = pltpu.make_async_remote_copy(
            src_ref=x_ref,
            dst_ref=xloc_ref.at[1],
            send_sem=sx_send,
            recv_sem=sx_recv,
            device_id=peer,
            device_id_type=pl.DeviceIdType.MESH,
        )
        ra = pltpu.make_async_remote_copy(
            src_ref=a_ref,
            dst_ref=aloc_ref.at[1],
            send_sem=sa_send,
            recv_sem=sa_recv,
            device_id=peer,
            device_id_type=pl.DeviceIdType.MESH,
        )
        rx.start()
        ra.start()
        rx.wait()
        ra.wait()

    return pl.pallas_call(
        body,
        out_shape=(
            jax.ShapeDtypeStruct((2, S, D), jnp.bfloat16),
            jax.ShapeDtypeStruct((2, S, 1), jnp.int32),
        ),
        in_specs=[
            pl.BlockSpec(memory_space=pltpu.MemorySpace.VMEM),
            pl.BlockSpec(memory_space=pltpu.MemorySpace.VMEM),
        ],
        out_specs=(
            pl.BlockSpec(memory_space=pltpu.MemorySpace.VMEM),
            pl.BlockSpec(memory_space=pltpu.MemorySpace.VMEM),
        ),
        scratch_shapes=[
            pltpu.SemaphoreType.DMA,
            pltpu.SemaphoreType.DMA,
            pltpu.SemaphoreType.DMA,
            pltpu.SemaphoreType.DMA,
        ],
    )(xsl, asl)


def _moe_kernel(xall, aall, W1f, W2f, FT=1024):
    n_m, S, D = xall.shape
    E_loc, _, F = W1f.shape
    n_f = F // FT

    def body(x_ref, a_ref, w1_ref, w2_ref, out_ref, acc_ref):
        e = pl.program_id(0)
        f = pl.program_id(1)
        m = pl.program_id(2)
        my_y = lax.axis_index("y")
        ge = my_y * E_loc + e

        mask = a_ref[0] == ge
        xm = jnp.where(mask, x_ref[0], jnp.bfloat16(0))
        w1 = w1_ref[0].astype(jnp.bfloat16)
        w2 = w2_ref[0].astype(jnp.bfloat16)
        h = jnp.dot(xm, w1, preferred_element_type=jnp.float32)
        h = jnp.maximum(h, 0.0).astype(jnp.bfloat16)
        part = jnp.dot(h, w2, preferred_element_type=jnp.float32)

        first = jnp.logical_and(e == 0, f == 0)
        last = jnp.logical_and(e == E_loc - 1, f == n_f - 1)

        @pl.when(first)
        def _():
            acc_ref[m] = part

        @pl.when(jnp.logical_not(first))
        def _():
            acc_ref[m] += part

        @pl.when(last)
        def _():
            out_ref[m] = acc_ref[m].astype(jnp.bfloat16)

    return pl.pallas_call(
        body,
        grid=(E_loc, n_f, n_m),
        out_shape=jax.ShapeDtypeStruct((n_m, S, D), jnp.bfloat16),
        in_specs=[
            pl.BlockSpec((1, S, D), lambda e, f, m: (m, 0, 0)),
            pl.BlockSpec((1, S, 1), lambda e, f, m: (m, 0, 0)),
            pl.BlockSpec((1, D, FT), lambda e, f, m: (e, 0, f)),
            pl.BlockSpec((1, FT, D), lambda e, f, m: (e, f, 0)),
        ],
        out_specs=pl.BlockSpec((n_m, S, D), lambda e, f, m: (0, 0, 0)),
        scratch_shapes=[pltpu.VMEM((n_m, S, D), jnp.float32)],
        compiler_params=pltpu.CompilerParams(
            dimension_semantics=("arbitrary", "arbitrary", "arbitrary"),
            vmem_limit_bytes=56 * 2**20,
        ),
    )(xall, aall, W1f, W2f)


def _combine_kernel(contrib):
    _, S, D = contrib.shape

    def body(c_ref, out_ref, recv_ref, copy_sem, send_sem, recv_sem):
        peer = _peer()
        local = pltpu.make_async_copy(c_ref.at[0], out_ref, copy_sem)
        local.start()
        rdma = pltpu.make_async_remote_copy(
            src_ref=c_ref.at[1],
            dst_ref=recv_ref,
            send_sem=send_sem,
            recv_sem=recv_sem,
            device_id=peer,
            device_id_type=pl.DeviceIdType.MESH,
        )
        rdma.start()
        local.wait()
        rdma.wait()
        out_ref[...] = out_ref[...] + recv_ref[...]

    return pl.pallas_call(
        body,
        out_shape=jax.ShapeDtypeStruct((S, D), jnp.bfloat16),
        in_specs=[pl.BlockSpec(memory_space=pltpu.MemorySpace.HBM)],
        out_specs=pl.BlockSpec(memory_space=pltpu.MemorySpace.VMEM),
        scratch_shapes=[
            pltpu.VMEM((S, D), jnp.bfloat16),
            pltpu.SemaphoreType.DMA,
            pltpu.SemaphoreType.DMA,
            pltpu.SemaphoreType.DMA,
        ],
    )(contrib)


def _allgather_kernel(chunk):
    S, D = chunk.shape
    n_r = N_RING // 2
    n_l = N_RING - 1 - n_r

    def body(c_ref, out_ref, commr_ref, comml_ref, sr_send, sr_recv, sl_send, sl_recv):
        rp = _ring_pos()
        my_y = lax.axis_index("y")
        rx, rz = _ring_coords((rp + 1) % N_RING)
        lx, lz = _ring_coords((rp - 1) % N_RING)
        right = (rx, my_y, rz)
        left = (lx, my_y, lz)

        out_ref[pl.ds(rp * S, S), :] = c_ref[...]
        commr_ref[0] = c_ref[...]
        comml_ref[0] = c_ref[...]

        for h in range(n_r):
            sslot = h % 2
            rslot = (h + 1) % 2
            rdma_r = pltpu.make_async_remote_copy(
                src_ref=commr_ref.at[sslot],
                dst_ref=commr_ref.at[rslot],
                send_sem=sr_send.at[sslot],
                recv_sem=sr_recv.at[rslot],
                device_id=right,
                device_id_type=pl.DeviceIdType.MESH,
            )
            rdma_r.start()
            if h < n_l:
                rdma_l = pltpu.make_async_remote_copy(
                    src_ref=comml_ref.at[sslot],
                    dst_ref=comml_ref.at[rslot],
                    send_sem=sl_send.at[sslot],
                    recv_sem=sl_recv.at[rslot],
                    device_id=left,
                    device_id_type=pl.DeviceIdType.MESH,
                )
                rdma_l.start()
            rdma_r.wait()
            origin_r = (rp - h - 1) % N_RING
            out_ref[pl.ds(origin_r * S, S), :] = commr_ref[rslot]
            if h < n_l:
                rdma_l.wait()
                origin_l = (rp + h + 1) % N_RING
                out_ref[pl.ds(origin_l * S, S), :] = comml_ref[rslot]

    return pl.pallas_call(
        body,
        out_shape=jax.ShapeDtypeStruct((N_RING * S, D), jnp.bfloat16),
        in_specs=[pl.BlockSpec(memory_space=pltpu.MemorySpace.VMEM)],
        out_specs=pl.BlockSpec(memory_space=pltpu.MemorySpace.VMEM),
        scratch_shapes=[
            pltpu.VMEM((2, S, D), jnp.bfloat16),
            pltpu.VMEM((2, S, D), jnp.bfloat16),
            pltpu.SemaphoreType.DMA((2,)),
            pltpu.SemaphoreType.DMA((2,)),
            pltpu.SemaphoreType.DMA((2,)),
            pltpu.SemaphoreType.DMA((2,)),
        ],
        compiler_params=pltpu.CompilerParams(vmem_limit_bytes=48 * 2**20),
    )(chunk)


def kernel(x, assign, W1, W2):
    T, D = x.shape
    S = T // N_RING
    xb = x.astype(jnp.bfloat16)
    a2 = assign.reshape(T, 1)

    rp = _ring_pos()
    xsl = lax.dynamic_slice(xb, (rp * S, 0), (S, D))
    asl = lax.dynamic_slice(a2, (rp * S, 0), (S, 1))

    xloc, aloc = _exchange_kernel(xsl, asl)
    contrib = _moe_kernel(xloc, aloc, W1, W2)
    mine = _combine_kernel(contrib)
    return _allgather_kernel(mine)


# device time: 396351 ns/iter; 7.6613x vs baseline; 1.0051x over previous
import jax
import jax.numpy as jnp
from jax import lax
from jax.experimental import pallas as pl
from jax.experimental.pallas import tpu as pltpu

N_RING = 8


def _peer():
    return (lax.axis_index("x"), 1 - lax.axis_index("y"), lax.axis_index("z"))


def _ring_pos():
    x = lax.axis_index("x")
    z = lax.axis_index("z")
    return jnp.where(x == 0, z, 2 * N_RING // 2 - 1 - z)


def _ring_coords(t):
    x = jnp.where(t < N_RING // 2, 0, 1)
    z = jnp.where(t < N_RING // 2, t, N_RING - 1 - t)
    return x, z


def _exchange_kernel(xsl, asl):
    S, D = xsl.shape

    def body(x_ref, a_ref, xloc_ref, aloc_ref, sx_send, sx_recv, sa_send, sa_recv):
        peer = _peer()
        xloc_ref[0] = x_ref[...]
        aloc_ref[0] = a_ref[...]
        rx = pltpu.make_async_remote_copy(
            src_ref=x_ref,
            dst_ref=xloc_ref.at[1],
            send_sem=sx_send,
            recv_sem=sx_recv,
            device_id=peer,
            device_id_type=pl.DeviceIdType.MESH,
        )
        ra = pltpu.make_async_remote_copy(
            src_ref=a_ref,
            dst_ref=aloc_ref.at[1],
            send_sem=sa_send,
            recv_sem=sa_recv,
            device_id=peer,
            device_id_type=pl.DeviceIdType.MESH,
        )
        rx.start()
        ra.start()
        rx.wait()
        ra.wait()

    return pl.pallas_call(
        body,
        out_shape=(
            jax.ShapeDtypeStruct((2, S, D), jnp.bfloat16),
            jax.ShapeDtypeStruct((2, S, 1), jnp.int32),
        ),
        in_specs=[
            pl.BlockSpec(memory_space=pltpu.MemorySpace.VMEM),
            pl.BlockSpec(memory_space=pltpu.MemorySpace.VMEM),
        ],
        out_specs=(
            pl.BlockSpec(memory_space=pltpu.MemorySpace.VMEM),
            pl.BlockSpec(memory_space=pltpu.MemorySpace.VMEM),
        ),
        scratch_shapes=[
            pltpu.SemaphoreType.DMA,
            pltpu.SemaphoreType.DMA,
            pltpu.SemaphoreType.DMA,
            pltpu.SemaphoreType.DMA,
        ],
    )(xsl, asl)


def _moe_kernel(xall, aall, W1f, W2f, FT=1024):
    n_m, S, D = xall.shape
    E_loc, _, F = W1f.shape
    n_f = F // FT

    def body(x_ref, a_ref, w1_ref, w2_ref, out_ref, acc_ref, xm_ref):
        e = pl.program_id(0)
        f = pl.program_id(1)
        m = pl.program_id(2)
        my_y = lax.axis_index("y")
        ge = my_y * E_loc + e

        @pl.when(f == 0)
        def _():
            mask = a_ref[0] == ge
            xm_ref[m] = jnp.where(mask, x_ref[0], jnp.bfloat16(0))

        w1 = w1_ref[0].astype(jnp.bfloat16)
        w2 = w2_ref[0].astype(jnp.bfloat16)
        h = jnp.dot(xm_ref[m], w1, preferred_element_type=jnp.float32)
        h = jnp.maximum(h, 0.0).astype(jnp.bfloat16)
        part = jnp.dot(h, w2, preferred_element_type=jnp.float32)

        first = jnp.logical_and(e == 0, f == 0)
        last = jnp.logical_and(e == E_loc - 1, f == n_f - 1)

        @pl.when(first)
        def _():
            acc_ref[m] = part

        @pl.when(jnp.logical_not(first))
        def _():
            acc_ref[m] += part

        @pl.when(last)
        def _():
            out_ref[m] = acc_ref[m].astype(jnp.bfloat16)

    return pl.pallas_call(
        body,
        grid=(E_loc, n_f, n_m),
        out_shape=jax.ShapeDtypeStruct((n_m, S, D), jnp.bfloat16),
        in_specs=[
            pl.BlockSpec((1, S, D), lambda e, f, m: (m, 0, 0)),
            pl.BlockSpec((1, S, 1), lambda e, f, m: (m, 0, 0)),
            pl.BlockSpec((1, D, FT), lambda e, f, m: (e, 0, f)),
            pl.BlockSpec((1, FT, D), lambda e, f, m: (e, f, 0)),
        ],
        out_specs=pl.BlockSpec((n_m, S, D), lambda e, f, m: (0, 0, 0)),
        scratch_shapes=[
            pltpu.VMEM((n_m, S, D), jnp.float32),
            pltpu.VMEM((n_m, S, D), jnp.bfloat16),
        ],
        compiler_params=pltpu.CompilerParams(
            dimension_semantics=("arbitrary", "arbitrary", "arbitrary"),
            vmem_limit_bytes=60 * 2**20,
        ),
    )(xall, aall, W1f, W2f)


def _combine_allgather_kernel(contrib):
    _, S, D = contrib.shape
    n_r = N_RING // 2
    n_l = N_RING - 1 - n_r

    def body(
        c_ref,
        out_ref,
        own_ref,
        crecv_ref,
        commr_ref,
        comml_ref,
        copy_sem,
        c_send,
        c_recv,
        sr_send,
        sr_recv,
        sl_send,
        sl_recv,
    ):
        rp = _ring_pos()
        my_y = lax.axis_index("y")
        peer = _peer()
        rx, rz = _ring_coords((rp + 1) % N_RING)
        lx, lz = _ring_coords((rp - 1) % N_RING)
        right = (rx, my_y, rz)
        left = (lx, my_y, lz)

        local = pltpu.make_async_copy(c_ref.at[0], own_ref, copy_sem)
        local.start()
        rdma_c = pltpu.make_async_remote_copy(
            src_ref=c_ref.at[1],
            dst_ref=crecv_ref,
            send_sem=c_send,
            recv_sem=c_recv,
            device_id=peer,
            device_id_type=pl.DeviceIdType.MESH,
        )
        rdma_c.start()
        local.wait()
        rdma_c.wait()
        own = own_ref[...] + crecv_ref[...]

        out_ref[pl.ds(rp * S, S), :] = own
        commr_ref[0] = own
        comml_ref[0] = own

        for h in range(n_r):
            sslot = h % 2
            rslot = (h + 1) % 2
            rdma_r = pltpu.make_async_remote_copy(
                src_ref=commr_ref.at[sslot],
                dst_ref=commr_ref.at[rslot],
                send_sem=sr_send.at[sslot],
                recv_sem=sr_recv.at[rslot],
                device_id=right,
                device_id_type=pl.DeviceIdType.MESH,
            )
            rdma_r.start()
            if h < n_l:
                rdma_l = pltpu.make_async_remote_copy(
                    src_ref=comml_ref.at[sslot],
                    dst_ref=comml_ref.at[rslot],
                    send_sem=sl_send.at[sslot],
                    recv_sem=sl_recv.at[rslot],
                    device_id=left,
                    device_id_type=pl.DeviceIdType.MESH,
                )
                rdma_l.start()
            rdma_r.wait()
            origin_r = (rp - h - 1) % N_RING
            out_ref[pl.ds(origin_r * S, S), :] = commr_ref[rslot]
            if h < n_l:
                rdma_l.wait()
                origin_l = (rp + h + 1) % N_RING
                out_ref[pl.ds(origin_l * S, S), :] = comml_ref[rslot]

    return pl.pallas_call(
        body,
        out_shape=jax.ShapeDtypeStruct((N_RING * S, D), jnp.bfloat16),
        in_specs=[pl.BlockSpec(memory_space=pltpu.MemorySpace.HBM)],
        out_specs=pl.BlockSpec(memory_space=pltpu.MemorySpace.VMEM),
        scratch_shapes=[
            pltpu.VMEM((S, D), jnp.bfloat16),
            pltpu.VMEM((S, D), jnp.bfloat16),
            pltpu.VMEM((2, S, D), jnp.bfloat16),
            pltpu.VMEM((2, S, D), jnp.bfloat16),
            pltpu.SemaphoreType.DMA,
            pltpu.SemaphoreType.DMA,
            pltpu.SemaphoreType.DMA,
            pltpu.SemaphoreType.DMA((2,)),
            pltpu.SemaphoreType.DMA((2,)),
            pltpu.SemaphoreType.DMA((2,)),
            pltpu.SemaphoreType.DMA((2,)),
        ],
        compiler_params=pltpu.CompilerParams(vmem_limit_bytes=48 * 2**20),
    )(contrib)


def kernel(x, assign, W1, W2):
    T, D = x.shape
    S = T // N_RING
    xb = x.astype(jnp.bfloat16)
    a2 = assign.reshape(T, 1)

    rp = _ring_pos()
    xsl = lax.dynamic_slice(xb, (rp * S, 0), (S, D))
    asl = lax.dynamic_slice(a2, (rp * S, 0), (S, 1))

    xloc, aloc = _exchange_kernel(xsl, asl)
    contrib = _moe_kernel(xloc, aloc, W1, W2)
    return _combine_allgather_kernel(contrib)


# device time: 365019 ns/iter; 8.3189x vs baseline; 1.0858x over previous
import jax
import jax.numpy as jnp
from jax import lax
from jax.experimental import pallas as pl
from jax.experimental.pallas import tpu as pltpu

N_RING = 8


def _peer():
    return (lax.axis_index("x"), 1 - lax.axis_index("y"), lax.axis_index("z"))


def _ring_pos():
    x = lax.axis_index("x")
    z = lax.axis_index("z")
    return jnp.where(x == 0, z, 2 * N_RING // 2 - 1 - z)


def _ring_coords(t):
    x = jnp.where(t < N_RING // 2, 0, 1)
    z = jnp.where(t < N_RING // 2, t, N_RING - 1 - t)
    return x, z


def _exchange_kernel(xsl, asl):
    S, D = xsl.shape

    def body(x_ref, a_ref, xloc_ref, aloc_ref, sx_send, sx_recv, sa_send, sa_recv):
        peer = _peer()
        xloc_ref[0] = x_ref[...]
        aloc_ref[0] = a_ref[...]
        rx = pltpu.make_async_remote_copy(
            src_ref=x_ref,
            dst_ref=xloc_ref.at[1],
            send_sem=sx_send,
            recv_sem=sx_recv,
            device_id=peer,
            device_id_type=pl.DeviceIdType.MESH,
        )
        ra = pltpu.make_async_remote_copy(
            src_ref=a_ref,
            dst_ref=aloc_ref.at[1],
            send_sem=sa_send,
            recv_sem=sa_recv,
            device_id=peer,
            device_id_type=pl.DeviceIdType.MESH,
        )
        rx.start()
        ra.start()
        rx.wait()
        ra.wait()

    return pl.pallas_call(
        body,
        out_shape=(
            jax.ShapeDtypeStruct((2, S, D), jnp.bfloat16),
            jax.ShapeDtypeStruct((2, S, 1), jnp.int32),
        ),
        in_specs=[
            pl.BlockSpec(memory_space=pltpu.MemorySpace.VMEM),
            pl.BlockSpec(memory_space=pltpu.MemorySpace.VMEM),
        ],
        out_specs=(
            pl.BlockSpec(memory_space=pltpu.MemorySpace.VMEM),
            pl.BlockSpec(memory_space=pltpu.MemorySpace.VMEM),
        ),
        scratch_shapes=[
            pltpu.SemaphoreType.DMA,
            pltpu.SemaphoreType.DMA,
            pltpu.SemaphoreType.DMA,
            pltpu.SemaphoreType.DMA,
        ],
    )(xsl, asl)


def _moe_kernel(xall, aall, W1f, W2f, FT=1024):
    n_m, S, D = xall.shape
    E_loc, _, F = W1f.shape
    n_f = F // FT
    R = n_m * S

    def body(x_ref, a_ref, w1_ref, w2_ref, out_ref, acc_ref, xm_ref):
        e = pl.program_id(0)
        f = pl.program_id(1)
        my_y = lax.axis_index("y")
        ge = my_y * E_loc + e

        @pl.when(f == 0)
        def _():
            mask = a_ref[...].reshape(R, 1) == ge
            xm_ref[...] = jnp.where(mask, x_ref[...].reshape(R, D), jnp.bfloat16(0))

        w1 = w1_ref[0].astype(jnp.bfloat16)
        w2 = w2_ref[0].astype(jnp.bfloat16)
        h = jnp.dot(xm_ref[...], w1, preferred_element_type=jnp.float32)
        h = jnp.maximum(h, 0.0).astype(jnp.bfloat16)
        part = jnp.dot(h, w2, preferred_element_type=jnp.float32)

        first = jnp.logical_and(e == 0, f == 0)
        last = jnp.logical_and(e == E_loc - 1, f == n_f - 1)

        @pl.when(first)
        def _():
            acc_ref[...] = part

        @pl.when(jnp.logical_not(first))
        def _():
            acc_ref[...] += part

        @pl.when(last)
        def _():
            out_ref[...] = acc_ref[...].astype(jnp.bfloat16).reshape(n_m, S, D)

    return pl.pallas_call(
        body,
        grid=(E_loc, n_f),
        out_shape=jax.ShapeDtypeStruct((n_m, S, D), jnp.bfloat16),
        in_specs=[
            pl.BlockSpec((n_m, S, D), lambda e, f: (0, 0, 0)),
            pl.BlockSpec((n_m, S, 1), lambda e, f: (0, 0, 0)),
            pl.BlockSpec((1, D, FT), lambda e, f: (e, 0, f)),
            pl.BlockSpec((1, FT, D), lambda e, f: (e, f, 0)),
        ],
        out_specs=pl.BlockSpec((n_m, S, D), lambda e, f: (0, 0, 0)),
        scratch_shapes=[
            pltpu.VMEM((R, D), jnp.float32),
            pltpu.VMEM((R, D), jnp.bfloat16),
        ],
        compiler_params=pltpu.CompilerParams(
            dimension_semantics=("arbitrary", "arbitrary"),
            vmem_limit_bytes=62 * 2**20,
        ),
    )(xall, aall, W1f, W2f)


def _combine_allgather_kernel(contrib):
    _, S, D = contrib.shape
    n_r = N_RING // 2
    n_l = N_RING - 1 - n_r

    def body(
        c_ref,
        out_ref,
        own_ref,
        crecv_ref,
        commr_ref,
        comml_ref,
        copy_sem,
        c_send,
        c_recv,
        sr_send,
        sr_recv,
        sl_send,
        sl_recv,
    ):
        rp = _ring_pos()
        my_y = lax.axis_index("y")
        peer = _peer()
        rx, rz = _ring_coords((rp + 1) % N_RING)
        lx, lz = _ring_coords((rp - 1) % N_RING)
        right = (rx, my_y, rz)
        left = (lx, my_y, lz)

        local = pltpu.make_async_copy(c_ref.at[0], own_ref, copy_sem)
        local.start()
        rdma_c = pltpu.make_async_remote_copy(
            src_ref=c_ref.at[1],
            dst_ref=crecv_ref,
            send_sem=c_send,
            recv_sem=c_recv,
            device_id=peer,
            device_id_type=pl.DeviceIdType.MESH,
        )
        rdma_c.start()
        local.wait()
        rdma_c.wait()
        own = own_ref[...] + crecv_ref[...]

        out_ref[pl.ds(rp * S, S), :] = own
        commr_ref[0] = own
        comml_ref[0] = own

        for h in range(n_r):
            sslot = h % 2
            rslot = (h + 1) % 2
            rdma_r = pltpu.make_async_remote_copy(
                src_ref=commr_ref.at[sslot],
                dst_ref=commr_ref.at[rslot],
                send_sem=sr_send.at[sslot],
                recv_sem=sr_recv.at[rslot],
                device_id=right,
                device_id_type=pl.DeviceIdType.MESH,
            )
            rdma_r.start()
            if h < n_l:
                rdma_l = pltpu.make_async_remote_copy(
                    src_ref=comml_ref.at[sslot],
                    dst_ref=comml_ref.at[rslot],
                    send_sem=sl_send.at[sslot],
                    recv_sem=sl_recv.at[rslot],
                    device_id=left,
                    device_id_type=pl.DeviceIdType.MESH,
                )
                rdma_l.start()
            rdma_r.wait()
            origin_r = (rp - h - 1) % N_RING
            out_ref[pl.ds(origin_r * S, S), :] = commr_ref[rslot]
            if h < n_l:
                rdma_l.wait()
                origin_l = (rp + h + 1) % N_RING
                out_ref[pl.ds(origin_l * S, S), :] = comml_ref[rslot]

    return pl.pallas_call(
        body,
        out_shape=jax.ShapeDtypeStruct((N_RING * S, D), jnp.bfloat16),
        in_specs=[pl.BlockSpec(memory_space=pltpu.MemorySpace.HBM)],
        out_specs=pl.BlockSpec(memory_space=pltpu.MemorySpace.VMEM),
        scratch_shapes=[
            pltpu.VMEM((S, D), jnp.bfloat16),
            pltpu.VMEM((S, D), jnp.bfloat16),
            pltpu.VMEM((2, S, D), jnp.bfloat16),
            pltpu.VMEM((2, S, D), jnp.bfloat16),
            pltpu.SemaphoreType.DMA,
            pltpu.SemaphoreType.DMA,
            pltpu.SemaphoreType.DMA,
            pltpu.SemaphoreType.DMA((2,)),
            pltpu.SemaphoreType.DMA((2,)),
            pltpu.SemaphoreType.DMA((2,)),
            pltpu.SemaphoreType.DMA((2,)),
        ],
        compiler_params=pltpu.CompilerParams(vmem_limit_bytes=48 * 2**20),
    )(contrib)


def kernel(x, assign, W1, W2):
    T, D = x.shape
    S = T // N_RING
    xb = x.astype(jnp.bfloat16)
    a2 = assign.reshape(T, 1)

    rp = _ring_pos()
    xsl = lax.dynamic_slice(xb, (rp * S, 0), (S, D))
    asl = lax.dynamic_slice(a2, (rp * S, 0), (S, 1))

    xloc, aloc = _exchange_kernel(xsl, asl)
    contrib = _moe_kernel(xloc, aloc, W1, W2)
    return _combine_allgather_kernel(contrib)
